# baseline (speedup 1.0000x reference)
import jax
import jax.numpy as jnp
from jax.experimental import pallas as pl
from jax.experimental.pallas import tpu as pltpu

_B, _N, _D = 8, 1024, 768
_P = 2048
_TN = 1024
_PC = 256
_ROWS = _B * _N
_GRID = _ROWS // _TN


def _top3_cols(d2):
    h = d2.shape[0] // 2
    a, b = d2[:h], d2[h:]
    p1 = jnp.minimum(a, b)
    p2 = jnp.maximum(a, b)
    h //= 2
    x1, x2 = p1[:h], p2[:h]
    y1, y2 = p1[h:], p2[h:]
    t1 = jnp.minimum(x1, y1)
    m = jnp.maximum(x1, y1)
    w = jnp.minimum(x2, y2)
    t2 = jnp.minimum(m, w)
    t3 = jnp.maximum(m, w)
    while h > 1:
        h //= 2
        t1, t2, t3 = _merge3((t1[:h], t2[:h], t3[:h]),
                             (t1[h:], t2[h:], t3[h:]))
    return t1, t2, t3


def _merge3(x, y):
    x1, x2, x3 = x
    y1, y2, y3 = y
    z1 = jnp.minimum(x1, y1)
    m = jnp.maximum(x1, y1)
    w = jnp.minimum(x2, y2)
    z2 = jnp.minimum(m, w)
    z3 = jnp.minimum(jnp.minimum(x3, y3),
                     jnp.minimum(jnp.maximum(x2, y1), jnp.maximum(x1, y2)))
    return z1, z2, z3


def _knn_body(x_ref, c_ref, o_ref):
    x = x_ref[...]
    tops = []
    for k in range(_P // _PC):
        c = c_ref[k * _PC:(k + 1) * _PC, :]
        cx = jax.lax.dot_general(
            c, x, (((1,), (1,)), ((), ())),
            preferred_element_type=jnp.float32,
        )
        cn = jnp.sum(c * c, axis=1, keepdims=True)
        q = cn - 2.0 * cx
        tops.append(_top3_cols(q))
    while len(tops) > 1:
        tops = [_merge3(tops[i], tops[i + 1]) for i in range(0, len(tops), 2)]
    t1, t2, t3 = tops[0]

    rn = jnp.sum(x * x, axis=1)[None, :]
    s1 = jnp.sqrt(t1 + rn)
    s2 = jnp.sqrt(t2 + rn)
    s3 = jnp.sqrt(t3 + rn)
    denom = 1.0 + jnp.exp(s1 - s2) + jnp.exp(s1 - s3)
    o_ref[0] = s1 / denom


def kernel(embeds, centroids):
    x = embeds.reshape(_ROWS, _D)
    out = pl.pallas_call(
        _knn_body,
        grid=(_GRID,),
        in_specs=[
            pl.BlockSpec((_TN, _D), lambda g: (g, 0)),
            pl.BlockSpec((_P, _D), lambda g: (0, 0)),
        ],
        out_specs=pl.BlockSpec((1, 1, _TN), lambda g: (g, 0, 0)),
        out_shape=jax.ShapeDtypeStruct((_GRID, 1, _TN), jnp.float32),
        compiler_params=pltpu.CompilerParams(
            dimension_semantics=("parallel",),
        ),
    )(x, centroids)
    return out.reshape(_B, 1, 32, 32)


# speedup vs baseline: 1.0568x; 1.0568x over previous
import jax
import jax.numpy as jnp
from jax.experimental import pallas as pl
from jax.experimental.pallas import tpu as pltpu

_B, _N, _D = 8, 1024, 768
_P = 2048
_TN = 2048
_PC = 256
_ROWS = _B * _N
_GRID = _ROWS // _TN


def _top3_cols(d2):
    h = d2.shape[0] // 2
    a, b = d2[:h], d2[h:]
    p1 = jnp.minimum(a, b)
    p2 = jnp.maximum(a, b)
    h //= 2
    x1, x2 = p1[:h], p2[:h]
    y1, y2 = p1[h:], p2[h:]
    t1 = jnp.minimum(x1, y1)
    m = jnp.maximum(x1, y1)
    w = jnp.minimum(x2, y2)
    t2 = jnp.minimum(m, w)
    t3 = jnp.maximum(m, w)
    while h > 1:
        h //= 2
        t1, t2, t3 = _merge3((t1[:h], t2[:h], t3[:h]),
                             (t1[h:], t2[h:], t3[h:]))
    return t1, t2, t3


def _merge3(x, y):
    x1, x2, x3 = x
    y1, y2, y3 = y
    z1 = jnp.minimum(x1, y1)
    m = jnp.maximum(x1, y1)
    w = jnp.minimum(x2, y2)
    z2 = jnp.minimum(m, w)
    z3 = jnp.minimum(jnp.minimum(x3, y3),
                     jnp.minimum(jnp.maximum(x2, y1), jnp.maximum(x1, y2)))
    return z1, z2, z3


def _knn_body(x_ref, c_ref, o_ref):
    x = x_ref[...]
    tops = []
    for k in range(_P // _PC):
        c = c_ref[k * _PC:(k + 1) * _PC, :]
        cx = jax.lax.dot_general(
            c, x, (((1,), (1,)), ((), ())),
            preferred_element_type=jnp.float32,
        )
        cn = jnp.sum(c * c, axis=1, keepdims=True)
        q = cn - 2.0 * cx
        tops.append(_top3_cols(q))
    while len(tops) > 1:
        tops = [_merge3(tops[i], tops[i + 1]) for i in range(0, len(tops), 2)]
    t1, t2, t3 = tops[0]

    rn = jnp.sum(x * x, axis=1)[None, :]
    s1 = jnp.sqrt(t1 + rn)
    s2 = jnp.sqrt(t2 + rn)
    s3 = jnp.sqrt(t3 + rn)
    denom = 1.0 + jnp.exp(s1 - s2) + jnp.exp(s1 - s3)
    o_ref[0] = s1 / denom


def kernel(embeds, centroids):
    x = embeds.reshape(_ROWS, _D)
    out = pl.pallas_call(
        _knn_body,
        grid=(_GRID,),
        in_specs=[
            pl.BlockSpec((_TN, _D), lambda g: (g, 0)),
            pl.BlockSpec((_P, _D), lambda g: (0, 0)),
        ],
        out_specs=pl.BlockSpec((1, 1, _TN), lambda g: (g, 0, 0)),
        out_shape=jax.ShapeDtypeStruct((_GRID, 1, _TN), jnp.float32),
        compiler_params=pltpu.CompilerParams(
            dimension_semantics=("parallel",),
        ),
    )(x, centroids)
    return out.reshape(_B, 1, 32, 32)
